# Initial kernel scaffold; baseline (speedup 1.0000x reference)
import jax
import jax.numpy as jnp
from jax.experimental import pallas as pl


def kernel(x, edge_index, W1, b1, W2, b2):
    raise NotImplementedError("write your pallas kernel here")


# speedup vs baseline: 17.9902x; 17.9902x over previous
import functools

import jax
import jax.numpy as jnp
from jax import lax
from jax.experimental import pallas as pl
from jax.experimental.pallas import tpu as pltpu
from jax.experimental.pallas import tpu_sc as plsc

N = 10000
E = 320000
NW = 32
EW = E // NW
CH = 128
NFULL = EW // CH
TAIL = EW - NFULL * CH
RPS = 624
REM = N - 16 * RPS


def _zero_rows(ref, nrows, ncols):
    z = jnp.zeros((16,), jnp.float32)

    def body(i, carry):
        for k in range(ncols // 16):
            ref[i, pl.ds(k * 16, 16)] = z
        return carry

    lax.fori_loop(0, nrows, body, 0)


def _make_agg(F):
    mesh = plsc.VectorSubcoreMesh(core_axis_name="c", subcore_axis_name="s")

    @functools.partial(
        pl.kernel,
        out_type=jax.ShapeDtypeStruct((2, N, F), jnp.float32),
        mesh=mesh,
        compiler_params=pltpu.CompilerParams(
            use_tc_tiling_on_sc=(F == 128)),
        scratch_types=[
            pltpu.VMEM((CH,), jnp.int32),
            pltpu.VMEM((CH,), jnp.int32),
            pltpu.VMEM((CH, F), jnp.float32),
            pltpu.VMEM((TAIL,), jnp.int32),
            pltpu.VMEM((TAIL,), jnp.int32),
            pltpu.VMEM((TAIL, F), jnp.float32),
            pltpu.VMEM_SHARED((N, F), jnp.float32),
            pltpu.SemaphoreType.DMA,
        ],
    )
    def agg(g_hbm, src_hbm, dst_hbm, out_hbm,
            src_v, dst_v, rows_v, srct_v, dstt_v, rowst_v, acc_sh, sem):
        c = lax.axis_index("c")
        s = lax.axis_index("s")
        wid = s * 2 + c
        _zero_rows(rows_v, CH, F)
        row0 = s * RPS
        for k in range(4):
            pltpu.sync_copy(rows_v, acc_sh.at[pl.ds(row0 + k * CH, CH)])
        pltpu.sync_copy(rows_v.at[pl.ds(0, RPS - 4 * CH)],
                        acc_sh.at[pl.ds(row0 + 4 * CH, RPS - 4 * CH)])

        @pl.when(s == 15)
        def _():
            pltpu.sync_copy(rows_v.at[pl.ds(0, REM)],
                            acc_sh.at[pl.ds(16 * RPS, REM)])

        plsc.subcore_barrier()

        ebase = wid * EW

        def body(j, carry):
            base = pl.multiple_of(ebase + j * CH, 8)
            pltpu.sync_copy(src_hbm.at[pl.ds(base, CH)], src_v)
            pltpu.sync_copy(dst_hbm.at[pl.ds(base, CH)], dst_v)
            pltpu.async_copy(g_hbm.at[src_v], rows_v, sem).wait()
            pltpu.sync_copy(rows_v, acc_sh.at[dst_v], add=True)
            return carry

        lax.fori_loop(0, NFULL, body, 0)

        tbase = pl.multiple_of(ebase + NFULL * CH, 8)
        pltpu.sync_copy(src_hbm.at[pl.ds(tbase, TAIL)], srct_v)
        pltpu.sync_copy(dst_hbm.at[pl.ds(tbase, TAIL)], dstt_v)
        pltpu.async_copy(g_hbm.at[srct_v], rowst_v, sem).wait()
        pltpu.sync_copy(rowst_v, acc_sh.at[dstt_v], add=True)

        plsc.subcore_barrier()
        pltpu.sync_copy(acc_sh.at[pl.ds(row0, RPS)],
                        out_hbm.at[c].at[pl.ds(row0, RPS)])

        @pl.when(s == 15)
        def _():
            pltpu.sync_copy(acc_sh.at[pl.ds(16 * RPS, REM)],
                            out_hbm.at[c].at[pl.ds(16 * RPS, REM)])

    return agg


def _make_deg():
    mesh = plsc.VectorSubcoreMesh(core_axis_name="c", subcore_axis_name="s")

    @functools.partial(
        pl.kernel,
        out_type=jax.ShapeDtypeStruct((2, N, 16), jnp.float32),
        mesh=mesh,
        compiler_params=pltpu.CompilerParams(use_tc_tiling_on_sc=False),
        scratch_types=[
            pltpu.VMEM((CH,), jnp.int32),
            pltpu.VMEM((CH, 16), jnp.float32),
            pltpu.VMEM((TAIL,), jnp.int32),
            pltpu.VMEM((TAIL, 16), jnp.float32),
            pltpu.VMEM_SHARED((N, 16), jnp.float32),
        ],
    )
    def deg(dst_hbm, out_hbm, dst_v, ones_v, dstt_v, onest_v, acc_sh):
        c = lax.axis_index("c")
        s = lax.axis_index("s")
        wid = s * 2 + c
        _zero_rows(ones_v, CH, 16)
        row0 = s * RPS
        for k in range(4):
            pltpu.sync_copy(ones_v, acc_sh.at[pl.ds(row0 + k * CH, CH)])
        pltpu.sync_copy(ones_v.at[pl.ds(0, RPS - 4 * CH)],
                        acc_sh.at[pl.ds(row0 + 4 * CH, RPS - 4 * CH)])

        @pl.when(s == 15)
        def _():
            pltpu.sync_copy(ones_v.at[pl.ds(0, REM)],
                            acc_sh.at[pl.ds(16 * RPS, REM)])

        one = jnp.ones((16,), jnp.float32)

        def fill(i, carry):
            ones_v[i, pl.ds(0, 16)] = one
            return carry

        lax.fori_loop(0, CH, fill, 0)

        def fillt(i, carry):
            onest_v[i, pl.ds(0, 16)] = one
            return carry

        lax.fori_loop(0, TAIL, fillt, 0)
        plsc.subcore_barrier()

        ebase = wid * EW

        def body(j, carry):
            base = pl.multiple_of(ebase + j * CH, 8)
            pltpu.sync_copy(dst_hbm.at[pl.ds(base, CH)], dst_v)
            pltpu.sync_copy(ones_v, acc_sh.at[dst_v], add=True)
            return carry

        lax.fori_loop(0, NFULL, body, 0)

        tbase = pl.multiple_of(ebase + NFULL * CH, 8)
        pltpu.sync_copy(dst_hbm.at[pl.ds(tbase, TAIL)], dstt_v)
        pltpu.sync_copy(onest_v, acc_sh.at[dstt_v], add=True)

        plsc.subcore_barrier()
        pltpu.sync_copy(acc_sh.at[pl.ds(row0, RPS)],
                        out_hbm.at[c].at[pl.ds(row0, RPS)])

        @pl.when(s == 15)
        def _():
            pltpu.sync_copy(acc_sh.at[pl.ds(16 * RPS, REM)],
                            out_hbm.at[c].at[pl.ds(16 * RPS, REM)])

    return deg


_R = 1000


def _tc1_body(degp, x, w1, g1, dinv16):
    deg = degp[0] + degp[1] + 1.0
    dinv = lax.rsqrt(deg)
    dinv16[...] = dinv
    g1[...] = dinv[:, 0:1] * jnp.dot(x[...], w1[...],
                                     preferred_element_type=jnp.float32)


def _tc2_body(aggp, g1, dinv16, b1, w2, g2):
    dinv = dinv16[:, 0:1]
    h = dinv * (aggp[0] + aggp[1] + g1[...]) + b1[...]
    h = jnp.maximum(h, 0.0)
    g2[...] = dinv * jnp.dot(h, w2[...], preferred_element_type=jnp.float32)


def _tc3_body(aggp, g2, dinv16, b2, out):
    dinv = dinv16[:, 0:1]
    logits = dinv * (aggp[0] + aggp[1] + g2[...]) + b2[...]
    m = jnp.max(logits, axis=1, keepdims=True)
    e = logits - m
    out[...] = e - jnp.log(jnp.sum(jnp.exp(e), axis=1, keepdims=True))


def _row_spec(F):
    return pl.BlockSpec((_R, F), lambda i: (i, 0))


def _pair_spec(F):
    return pl.BlockSpec((2, _R, F), lambda i: (0, i, 0))


def _full_spec(a, b):
    return pl.BlockSpec((a, b), lambda i: (0, 0))


_tc1 = pl.pallas_call(
    _tc1_body,
    grid=(N // _R,),
    in_specs=[_pair_spec(16), _row_spec(128), _full_spec(128, 128)],
    out_specs=[_row_spec(128), _row_spec(16)],
    out_shape=[jax.ShapeDtypeStruct((N, 128), jnp.float32),
               jax.ShapeDtypeStruct((N, 16), jnp.float32)],
)

_tc2 = pl.pallas_call(
    _tc2_body,
    grid=(N // _R,),
    in_specs=[_pair_spec(128), _row_spec(128), _row_spec(16),
              _full_spec(1, 128), _full_spec(128, 64)],
    out_specs=[_row_spec(64)],
    out_shape=[jax.ShapeDtypeStruct((N, 64), jnp.float32)],
)

_tc3 = pl.pallas_call(
    _tc3_body,
    grid=(N // _R,),
    in_specs=[_pair_spec(64), _row_spec(64), _row_spec(16),
              _full_spec(1, 64)],
    out_specs=[_row_spec(64)],
    out_shape=[jax.ShapeDtypeStruct((N, 64), jnp.float32)],
)

_agg128 = _make_agg(128)
_agg64 = _make_agg(64)
_deg = _make_deg()


@jax.jit
def kernel(x, edge_index, W1, b1, W2, b2):
    src = edge_index[0]
    dst = edge_index[1]
    degp = _deg(dst)
    g1, dinv16 = _tc1(degp, x, W1)
    aggp1 = _agg128(g1, src, dst)
    (g2,) = _tc2(aggp1, g1, dinv16, b1.reshape(1, -1), W2)
    aggp2 = _agg64(g2, src, dst)
    (out,) = _tc3(aggp2, g2, dinv16, b2.reshape(1, -1))
    return out


# speedup vs baseline: 27.4554x; 1.5261x over previous
import functools

import jax
import jax.numpy as jnp
from jax import lax
from jax.experimental import pallas as pl
from jax.experimental.pallas import tpu as pltpu
from jax.experimental.pallas import tpu_sc as plsc

N = 10000
E = 320000
CH = 128
NCHUNK = E // CH
NW = 32
CPW = NCHUNK // NW
XTRA = NCHUNK - CPW * NW
RPS = 624
REM = N - 16 * RPS

_SC_PARAMS = pltpu.CompilerParams(use_tc_tiling_on_sc=False)


def _zero_rows(ref, nrows, ncols):
    z = jnp.zeros((16,), jnp.float32)

    def body(i, carry):
        for k in range(ncols // 16):
            ref[i, pl.ds(k * 16, 16)] = z
        return carry

    lax.fori_loop(0, nrows, body, 0)


def _zero_acc_slice(zbuf, acc_sh, s):
    row0 = s * RPS
    for k in range(4):
        pltpu.sync_copy(zbuf, acc_sh.at[pl.ds(row0 + k * CH, CH)])
    pltpu.sync_copy(zbuf.at[pl.ds(0, RPS - 4 * CH)],
                    acc_sh.at[pl.ds(row0 + 4 * CH, RPS - 4 * CH)])

    @pl.when(s == 15)
    def _():
        pltpu.sync_copy(zbuf.at[pl.ds(0, REM)],
                        acc_sh.at[pl.ds(16 * RPS, REM)])


def _copy_out(acc_sh, out_hbm, c, s):
    row0 = s * RPS
    pltpu.sync_copy(acc_sh.at[pl.ds(row0, RPS)],
                    out_hbm.at[c].at[pl.ds(row0, RPS)])

    @pl.when(s == 15)
    def _():
        pltpu.sync_copy(acc_sh.at[pl.ds(16 * RPS, REM)],
                        out_hbm.at[c].at[pl.ds(16 * RPS, REM)])


def _make_agg(F):
    mesh = plsc.VectorSubcoreMesh(core_axis_name="c", subcore_axis_name="s")

    @functools.partial(
        pl.kernel,
        out_type=jax.ShapeDtypeStruct((2, N, F), jnp.float32),
        mesh=mesh,
        compiler_params=_SC_PARAMS,
        scratch_types=[
            pltpu.VMEM((CH,), jnp.int32),
            pltpu.VMEM((CH,), jnp.int32),
            pltpu.VMEM((1, CH), jnp.int32),
            pltpu.VMEM((1, CH), jnp.int32),
            pltpu.VMEM((CH, F), jnp.float32),
            pltpu.VMEM((CH, F), jnp.float32),
            pltpu.VMEM_SHARED((N, F), jnp.float32),
            pltpu.SemaphoreType.DMA,
            pltpu.SemaphoreType.DMA,
        ],
    )
    def agg(g_hbm, src_hbm, dst2_hbm, out_hbm,
            sbuf0, sbuf1, dbuf0, dbuf1, rows0_v, rows1_v, acc_sh,
            sem0, sem1):
        c = lax.axis_index("c")
        s = lax.axis_index("s")
        wid = s * 2 + c
        cstart = wid * CPW + jnp.minimum(wid, XTRA)
        extra = wid < XTRA

        _zero_rows(rows0_v, CH, F)
        _zero_acc_slice(rows0_v, acc_sh, s)
        plsc.subcore_barrier()

        sbufs = (sbuf0, sbuf1)
        dbufs = (dbuf0, dbuf1)
        rows = (rows0_v, rows1_v)
        sems = (sem0, sem1)

        def load_idx(j, b):
            pltpu.sync_copy(src_hbm.at[pl.ds((cstart + j) * CH, CH)],
                            sbufs[b])
            pltpu.sync_copy(dst2_hbm.at[pl.ds(cstart + j, 1)], dbufs[b])

        def gather(b):
            return pltpu.async_copy(g_hbm.at[sbufs[b]], rows[b], sems[b])

        def wait_gather(b):
            pltpu.make_async_copy(g_hbm.at[sbufs[b]], rows[b],
                                  sems[b]).wait()

        def scatter(b):
            pltpu.sync_copy(rows[b], acc_sh.at[dbufs[b].at[0]], add=True)

        load_idx(0, 0)
        gather(0)

        def body(k, carry):
            j = k * 2
            load_idx(j + 1, 1)
            gather(1)
            wait_gather(0)
            scatter(0)

            @pl.when(j + 2 < CPW)
            def _():
                load_idx(j + 2, 0)
                gather(0)

            wait_gather(1)
            scatter(1)
            return carry

        lax.fori_loop(0, CPW // 2, body, 0)

        @pl.when(extra)
        def _():
            load_idx(CPW, 0)
            gather(0).wait()
            scatter(0)

        plsc.subcore_barrier()
        _copy_out(acc_sh, out_hbm, c, s)

    return agg


def _make_deg():
    mesh = plsc.VectorSubcoreMesh(core_axis_name="c", subcore_axis_name="s")

    @functools.partial(
        pl.kernel,
        out_type=jax.ShapeDtypeStruct((2, N, 16), jnp.float32),
        mesh=mesh,
        compiler_params=_SC_PARAMS,
        scratch_types=[
            pltpu.VMEM((CPW + 1, CH), jnp.int32),
            pltpu.VMEM((CH, 16), jnp.float32),
            pltpu.VMEM_SHARED((N, 16), jnp.float32),
        ],
    )
    def deg(dst2_hbm, out_hbm, dstr_v, ones_v, acc_sh):
        c = lax.axis_index("c")
        s = lax.axis_index("s")
        wid = s * 2 + c
        cstart = wid * CPW + jnp.minimum(wid, XTRA)
        extra = wid < XTRA

        _zero_rows(ones_v, CH, 16)
        _zero_acc_slice(ones_v, acc_sh, s)
        pltpu.sync_copy(dst2_hbm.at[pl.ds(cstart, CPW)],
                        dstr_v.at[pl.ds(0, CPW)])

        @pl.when(extra)
        def _():
            pltpu.sync_copy(dst2_hbm.at[pl.ds(cstart + CPW, 1)],
                            dstr_v.at[pl.ds(CPW, 1)])

        one = jnp.ones((16,), jnp.float32)

        def fill(i, carry):
            ones_v[i, pl.ds(0, 16)] = one
            return carry

        lax.fori_loop(0, CH, fill, 0)
        plsc.subcore_barrier()

        def body(j, carry):
            pltpu.sync_copy(ones_v, acc_sh.at[dstr_v.at[j]], add=True)
            return carry

        lax.fori_loop(0, CPW, body, 0)

        @pl.when(extra)
        def _():
            pltpu.sync_copy(ones_v, acc_sh.at[dstr_v.at[CPW]], add=True)

        plsc.subcore_barrier()
        _copy_out(acc_sh, out_hbm, c, s)

    return deg


_R = 1000


def _tc1_body(degp, x, w1, g1, dinv16):
    deg = degp[0] + degp[1] + 1.0
    dinv = lax.rsqrt(deg)
    dinv16[...] = dinv
    g1[...] = dinv[:, 0:1] * jnp.dot(x[...], w1[...],
                                     preferred_element_type=jnp.float32)


def _tc2_body(aggp, g1, dinv16, b1, w2, g2):
    dinv = dinv16[:, 0:1]
    h = dinv * (aggp[0] + aggp[1] + g1[...]) + b1[...]
    h = jnp.maximum(h, 0.0)
    g2[...] = dinv * jnp.dot(h, w2[...], preferred_element_type=jnp.float32)


def _tc3_body(aggp, g2, dinv16, b2, out):
    dinv = dinv16[:, 0:1]
    logits = dinv * (aggp[0] + aggp[1] + g2[...]) + b2[...]
    m = jnp.max(logits, axis=1, keepdims=True)
    e = logits - m
    out[...] = e - jnp.log(jnp.sum(jnp.exp(e), axis=1, keepdims=True))


def _row_spec(F):
    return pl.BlockSpec((_R, F), lambda i: (i, 0))


def _pair_spec(F):
    return pl.BlockSpec((2, _R, F), lambda i: (0, i, 0))


def _full_spec(a, b):
    return pl.BlockSpec((a, b), lambda i: (0, 0))


_tc1 = pl.pallas_call(
    _tc1_body,
    grid=(N // _R,),
    in_specs=[_pair_spec(16), _row_spec(128), _full_spec(128, 128)],
    out_specs=[_row_spec(128), _row_spec(16)],
    out_shape=[jax.ShapeDtypeStruct((N, 128), jnp.float32),
               jax.ShapeDtypeStruct((N, 16), jnp.float32)],
)

_tc2 = pl.pallas_call(
    _tc2_body,
    grid=(N // _R,),
    in_specs=[_pair_spec(128), _row_spec(128), _row_spec(16),
              _full_spec(1, 128), _full_spec(128, 64)],
    out_specs=[_row_spec(64)],
    out_shape=[jax.ShapeDtypeStruct((N, 64), jnp.float32)],
)

_tc3 = pl.pallas_call(
    _tc3_body,
    grid=(N // _R,),
    in_specs=[_pair_spec(64), _row_spec(64), _row_spec(16),
              _full_spec(1, 64)],
    out_specs=[_row_spec(64)],
    out_shape=[jax.ShapeDtypeStruct((N, 64), jnp.float32)],
)

_agg128 = _make_agg(128)
_agg64 = _make_agg(64)
_deg = _make_deg()


@jax.jit
def kernel(x, edge_index, W1, b1, W2, b2):
    src = edge_index[0]
    dst = edge_index[1]
    dst2 = dst.reshape(NCHUNK, CH)
    degp = _deg(dst2)
    g1, dinv16 = _tc1(degp, x, W1)
    aggp1 = _agg128(g1, src, dst2)
    (g2,) = _tc2(aggp1, g1, dinv16, b1.reshape(1, -1), W2)
    aggp2 = _agg64(g2, src, dst2)
    (out,) = _tc3(aggp2, g2, dinv16, b2.reshape(1, -1))
    return out


# speedup vs baseline: 33.2880x; 1.2124x over previous
import functools

import jax
import jax.numpy as jnp
from jax import lax
from jax.experimental import pallas as pl
from jax.experimental.pallas import tpu as pltpu
from jax.experimental.pallas import tpu_sc as plsc

N = 10000
E = 320000
CH = 128
NCHUNK = E // CH
NW = 32
CPW = NCHUNK // NW
XTRA = NCHUNK - CPW * NW
GS = 6
NG = CPW // GS
RPS = 624
REM = N - 16 * RPS

_SC_PARAMS = pltpu.CompilerParams(use_tc_tiling_on_sc=False)


def _zero_rows(ref, nrows, ncols):
    z = jnp.zeros((16,), jnp.float32)

    def body(i, carry):
        for k in range(ncols // 16):
            ref[i, pl.ds(k * 16, 16)] = z
        return carry

    lax.fori_loop(0, nrows, body, 0)


def _zero_acc_slice(zbuf, acc_sh, s):
    row0 = s * RPS
    for k in range(4):
        pltpu.sync_copy(zbuf, acc_sh.at[pl.ds(row0 + k * CH, CH)])
    pltpu.sync_copy(zbuf.at[pl.ds(0, RPS - 4 * CH)],
                    acc_sh.at[pl.ds(row0 + 4 * CH, RPS - 4 * CH)])

    @pl.when(s == 15)
    def _():
        pltpu.sync_copy(zbuf.at[pl.ds(0, REM)],
                        acc_sh.at[pl.ds(16 * RPS, REM)])


def _copy_out(acc_sh, out_hbm, c, s):
    row0 = s * RPS
    pltpu.sync_copy(acc_sh.at[pl.ds(row0, RPS)],
                    out_hbm.at[c].at[pl.ds(row0, RPS)])

    @pl.when(s == 15)
    def _():
        pltpu.sync_copy(acc_sh.at[pl.ds(16 * RPS, REM)],
                        out_hbm.at[c].at[pl.ds(16 * RPS, REM)])


def _make_agg(F):
    mesh = plsc.VectorSubcoreMesh(core_axis_name="c", subcore_axis_name="s")

    @functools.partial(
        pl.kernel,
        out_type=jax.ShapeDtypeStruct((2, N, F), jnp.float32),
        mesh=mesh,
        compiler_params=_SC_PARAMS,
        scratch_types=[
            pltpu.VMEM((GS * CH,), jnp.int32),
            pltpu.VMEM((GS * CH,), jnp.int32),
            pltpu.VMEM((GS, CH), jnp.int32),
            pltpu.VMEM((GS, CH), jnp.int32),
            pltpu.VMEM((CH, F), jnp.float32),
            pltpu.VMEM((CH, F), jnp.float32),
            pltpu.VMEM_SHARED((N, F), jnp.float32),
            pltpu.SemaphoreType.DMA,
            pltpu.SemaphoreType.DMA,
        ],
    )
    def agg(g_hbm, src_hbm, dst2_hbm, out_hbm,
            sbuf0, sbuf1, dbuf0, dbuf1, rows0_v, rows1_v, acc_sh,
            sem0, sem1):
        c = lax.axis_index("c")
        s = lax.axis_index("s")
        wid = s * 2 + c
        cstart = wid * CPW + jnp.minimum(wid, XTRA)
        extra = wid < XTRA

        _zero_rows(rows0_v, CH, F)
        _zero_acc_slice(rows0_v, acc_sh, s)
        plsc.subcore_barrier()

        sbufs = (sbuf0, sbuf1)
        dbufs = (dbuf0, dbuf1)
        rows = (rows0_v, rows1_v)
        sems = (sem0, sem1)

        def load_group(g, p):
            pltpu.sync_copy(src_hbm.at[pl.ds((cstart + g * GS) * CH,
                                             GS * CH)], sbufs[p])
            pltpu.sync_copy(dst2_hbm.at[pl.ds(cstart + g * GS, GS)],
                            dbufs[p])

        def gather(p, t, b):
            return pltpu.async_copy(
                g_hbm.at[sbufs[p].at[pl.ds(t * CH, CH)]], rows[b], sems[b])

        def wait_gather(b):
            pltpu.make_async_copy(g_hbm.at[sbufs[0].at[pl.ds(0, CH)]],
                                  rows[b], sems[b]).wait()

        def scatter(p, t, b):
            pltpu.sync_copy(rows[b], acc_sh.at[dbufs[p].at[t]], add=True)

        def process(p, t, nxt):
            b = t % 2
            if nxt is not None:
                gather(nxt[0], nxt[1], 1 - b)
            wait_gather(b)
            scatter(p, t, b)

        def do_group(p, next_p):
            for t in range(GS):
                nxt = (p, t + 1) if t < GS - 1 else (next_p, 0)
                process(p, t, nxt)

        load_group(0, 0)
        load_group(1, 1)
        gather(0, 0, 0)

        def body(k, carry):
            do_group(0, 1)
            load_group(2 * k + 2, 0)
            do_group(1, 0)

            @pl.when(k < NG // 2 - 1)
            def _():
                load_group(2 * k + 3, 1)

            return carry

        lax.fori_loop(0, NG // 2, body, 0)

        for t in range(GS):
            process(0, t, (0, t + 1) if t < GS - 1 else None)

        @pl.when(extra)
        def _():
            pltpu.sync_copy(src_hbm.at[pl.ds((cstart + CPW) * CH, CH)],
                            sbufs[1].at[pl.ds(0, CH)])
            pltpu.sync_copy(dst2_hbm.at[pl.ds(cstart + CPW, 1)],
                            dbufs[1].at[pl.ds(0, 1)])
            gather(1, 0, 0).wait()
            scatter(1, 0, 0)

        plsc.subcore_barrier()
        _copy_out(acc_sh, out_hbm, c, s)

    return agg


def _make_deg():
    mesh = plsc.VectorSubcoreMesh(core_axis_name="c", subcore_axis_name="s")

    @functools.partial(
        pl.kernel,
        out_type=jax.ShapeDtypeStruct((2, N, 16), jnp.float32),
        mesh=mesh,
        compiler_params=_SC_PARAMS,
        scratch_types=[
            pltpu.VMEM((CPW + 1, CH), jnp.int32),
            pltpu.VMEM((CH, 16), jnp.float32),
            pltpu.VMEM_SHARED((N, 16), jnp.float32),
            pltpu.SemaphoreType.DMA,
        ],
    )
    def deg(dst2_hbm, out_hbm, dstr_v, ones_v, acc_sh, ssem):
        c = lax.axis_index("c")
        s = lax.axis_index("s")
        wid = s * 2 + c
        cstart = wid * CPW + jnp.minimum(wid, XTRA)
        extra = wid < XTRA

        _zero_rows(ones_v, CH, 16)
        _zero_acc_slice(ones_v, acc_sh, s)
        pltpu.sync_copy(dst2_hbm.at[pl.ds(cstart, CPW)],
                        dstr_v.at[pl.ds(0, CPW)])

        @pl.when(extra)
        def _():
            pltpu.sync_copy(dst2_hbm.at[pl.ds(cstart + CPW, 1)],
                            dstr_v.at[pl.ds(CPW, 1)])

        one = jnp.ones((16,), jnp.float32)

        def fill(i, carry):
            ones_v[i, pl.ds(0, 16)] = one
            return carry

        lax.fori_loop(0, CH, fill, 0)
        plsc.subcore_barrier()

        DGRP = 13

        def body(k, carry):
            for t in range(DGRP):
                pltpu.make_async_copy(
                    ones_v, acc_sh.at[dstr_v.at[k * DGRP + t]],
                    ssem).start(add=True)
            for t in range(DGRP):
                pltpu.make_async_copy(
                    ones_v, acc_sh.at[dstr_v.at[k * DGRP + t]],
                    ssem).wait()
            return carry

        lax.fori_loop(0, CPW // DGRP, body, 0)

        @pl.when(extra)
        def _():
            pltpu.sync_copy(ones_v, acc_sh.at[dstr_v.at[CPW]], add=True)

        plsc.subcore_barrier()
        _copy_out(acc_sh, out_hbm, c, s)

    return deg


_R = 1000


def _tc1_body(degp, x, w1, g1, dinv16):
    deg = degp[0] + degp[1] + 1.0
    dinv = lax.rsqrt(deg)
    dinv16[...] = dinv
    g1[...] = dinv[:, 0:1] * jnp.dot(x[...], w1[...],
                                     preferred_element_type=jnp.float32)


def _tc2_body(aggp, g1, dinv16, b1, w2, g2):
    dinv = dinv16[:, 0:1]
    h = dinv * (aggp[0] + aggp[1] + g1[...]) + b1[...]
    h = jnp.maximum(h, 0.0)
    g2[...] = dinv * jnp.dot(h, w2[...], preferred_element_type=jnp.float32)


def _tc3_body(aggp, g2, dinv16, b2, out):
    dinv = dinv16[:, 0:1]
    logits = dinv * (aggp[0] + aggp[1] + g2[...]) + b2[...]
    m = jnp.max(logits, axis=1, keepdims=True)
    e = logits - m
    out[...] = e - jnp.log(jnp.sum(jnp.exp(e), axis=1, keepdims=True))


def _row_spec(F):
    return pl.BlockSpec((_R, F), lambda i: (i, 0))


def _pair_spec(F):
    return pl.BlockSpec((2, _R, F), lambda i: (0, i, 0))


def _full_spec(a, b):
    return pl.BlockSpec((a, b), lambda i: (0, 0))


_tc1 = pl.pallas_call(
    _tc1_body,
    grid=(N // _R,),
    in_specs=[_pair_spec(16), _row_spec(128), _full_spec(128, 128)],
    out_specs=[_row_spec(128), _row_spec(16)],
    out_shape=[jax.ShapeDtypeStruct((N, 128), jnp.float32),
               jax.ShapeDtypeStruct((N, 16), jnp.float32)],
)

_tc2 = pl.pallas_call(
    _tc2_body,
    grid=(N // _R,),
    in_specs=[_pair_spec(128), _row_spec(128), _row_spec(16),
              _full_spec(1, 128), _full_spec(128, 64)],
    out_specs=[_row_spec(64)],
    out_shape=[jax.ShapeDtypeStruct((N, 64), jnp.float32)],
)

_tc3 = pl.pallas_call(
    _tc3_body,
    grid=(N // _R,),
    in_specs=[_pair_spec(64), _row_spec(64), _row_spec(16),
              _full_spec(1, 64)],
    out_specs=[_row_spec(64)],
    out_shape=[jax.ShapeDtypeStruct((N, 64), jnp.float32)],
)

_agg128 = _make_agg(128)
_agg64 = _make_agg(64)
_deg = _make_deg()


@jax.jit
def kernel(x, edge_index, W1, b1, W2, b2):
    src = edge_index[0]
    dst = edge_index[1]
    dst2 = dst.reshape(NCHUNK, CH)
    degp = _deg(dst2)
    g1, dinv16 = _tc1(degp, x, W1)
    aggp1 = _agg128(g1, src, dst2)
    (g2,) = _tc2(aggp1, g1, dinv16, b1.reshape(1, -1), W2)
    aggp2 = _agg64(g2, src, dst2)
    (out,) = _tc3(aggp2, g2, dinv16, b2.reshape(1, -1))
    return out


# speedup vs baseline: 34.4700x; 1.0355x over previous
import functools

import jax
import jax.numpy as jnp
from jax import lax
from jax.experimental import pallas as pl
from jax.experimental.pallas import tpu as pltpu
from jax.experimental.pallas import tpu_sc as plsc

N = 10000
E = 320000
CH = 128
NCHUNK = E // CH
NW = 32
CPW = NCHUNK // NW
XTRA = NCHUNK - CPW * NW
GS = 6
NG = CPW // GS
RPS = 624
REM = N - 16 * RPS

_SC_PARAMS = pltpu.CompilerParams(use_tc_tiling_on_sc=False)


def _zero_rows(ref, nrows, ncols):
    z = jnp.zeros((16,), jnp.float32)

    def body(i, carry):
        for k in range(ncols // 16):
            ref[i, pl.ds(k * 16, 16)] = z
        return carry

    lax.fori_loop(0, nrows, body, 0)


def _zero_acc_slice(zbuf, acc_sh, s):
    row0 = s * RPS
    for k in range(4):
        pltpu.sync_copy(zbuf, acc_sh.at[pl.ds(row0 + k * CH, CH)])
    pltpu.sync_copy(zbuf.at[pl.ds(0, RPS - 4 * CH)],
                    acc_sh.at[pl.ds(row0 + 4 * CH, RPS - 4 * CH)])

    @pl.when(s == 15)
    def _():
        pltpu.sync_copy(zbuf.at[pl.ds(0, REM)],
                        acc_sh.at[pl.ds(16 * RPS, REM)])


def _copy_out(acc_sh, out_hbm, c, s):
    row0 = s * RPS
    pltpu.sync_copy(acc_sh.at[pl.ds(row0, RPS)],
                    out_hbm.at[c].at[pl.ds(row0, RPS)])

    @pl.when(s == 15)
    def _():
        pltpu.sync_copy(acc_sh.at[pl.ds(16 * RPS, REM)],
                        out_hbm.at[c].at[pl.ds(16 * RPS, REM)])


def _make_agg(F):
    mesh = plsc.VectorSubcoreMesh(core_axis_name="c", subcore_axis_name="s")

    @functools.partial(
        pl.kernel,
        out_type=jax.ShapeDtypeStruct((2, N, F), jnp.float32),
        mesh=mesh,
        compiler_params=_SC_PARAMS,
        scratch_types=[
            pltpu.VMEM((GS, CH), jnp.int32),
            pltpu.VMEM((GS, CH), jnp.int32),
            pltpu.VMEM((GS, CH), jnp.int32),
            pltpu.VMEM((GS, CH), jnp.int32),
            pltpu.VMEM((CH, F), jnp.float32),
            pltpu.VMEM((CH, F), jnp.float32),
            pltpu.VMEM_SHARED((N, F), jnp.float32),
            pltpu.SemaphoreType.DMA,
            pltpu.SemaphoreType.DMA,
        ],
    )
    def agg(g_hbm, ei2_hbm, out_hbm,
            sbuf0, sbuf1, dbuf0, dbuf1, rows0_v, rows1_v, acc_sh,
            sem0, sem1):
        c = lax.axis_index("c")
        s = lax.axis_index("s")
        wid = s * 2 + c
        cstart = wid * CPW + jnp.minimum(wid, XTRA)
        extra = wid < XTRA

        _zero_rows(rows0_v, CH, F)
        _zero_acc_slice(rows0_v, acc_sh, s)
        plsc.subcore_barrier()

        sbufs = (sbuf0, sbuf1)
        dbufs = (dbuf0, dbuf1)
        rows = (rows0_v, rows1_v)
        sems = (sem0, sem1)

        def load_group(g, p):
            pltpu.sync_copy(ei2_hbm.at[0].at[pl.ds(cstart + g * GS, GS)],
                            sbufs[p])
            pltpu.sync_copy(ei2_hbm.at[1].at[pl.ds(cstart + g * GS, GS)],
                            dbufs[p])

        def gather(p, t, b):
            return pltpu.async_copy(
                g_hbm.at[sbufs[p].at[t]], rows[b], sems[b])

        def wait_gather(b):
            pltpu.make_async_copy(g_hbm.at[sbufs[0].at[0]],
                                  rows[b], sems[b]).wait()

        def scatter(p, t, b):
            pltpu.sync_copy(rows[b], acc_sh.at[dbufs[p].at[t]], add=True)

        def process(p, t, nxt):
            b = t % 2
            if nxt is not None:
                gather(nxt[0], nxt[1], 1 - b)
            wait_gather(b)
            scatter(p, t, b)

        def do_group(p, next_p):
            for t in range(GS):
                nxt = (p, t + 1) if t < GS - 1 else (next_p, 0)
                process(p, t, nxt)

        load_group(0, 0)
        load_group(1, 1)
        gather(0, 0, 0)

        def body(k, carry):
            do_group(0, 1)
            load_group(2 * k + 2, 0)
            do_group(1, 0)

            @pl.when(k < NG // 2 - 1)
            def _():
                load_group(2 * k + 3, 1)

            return carry

        lax.fori_loop(0, NG // 2, body, 0)

        for t in range(GS):
            process(0, t, (0, t + 1) if t < GS - 1 else None)

        @pl.when(extra)
        def _():
            pltpu.sync_copy(ei2_hbm.at[0].at[pl.ds(cstart + CPW, 1)],
                            sbufs[1].at[pl.ds(0, 1)])
            pltpu.sync_copy(ei2_hbm.at[1].at[pl.ds(cstart + CPW, 1)],
                            dbufs[1].at[pl.ds(0, 1)])
            gather(1, 0, 0).wait()
            scatter(1, 0, 0)

        plsc.subcore_barrier()
        _copy_out(acc_sh, out_hbm, c, s)

    return agg


def _make_deg():
    mesh = plsc.VectorSubcoreMesh(core_axis_name="c", subcore_axis_name="s")

    @functools.partial(
        pl.kernel,
        out_type=jax.ShapeDtypeStruct((2, N, 16), jnp.float32),
        mesh=mesh,
        compiler_params=_SC_PARAMS,
        scratch_types=[
            pltpu.VMEM((CPW + 1, CH), jnp.int32),
            pltpu.VMEM((CH, 16), jnp.float32),
            pltpu.VMEM_SHARED((N, 16), jnp.float32),
            pltpu.SemaphoreType.DMA,
        ],
    )
    def deg(ei2_hbm, out_hbm, dstr_v, ones_v, acc_sh, ssem):
        c = lax.axis_index("c")
        s = lax.axis_index("s")
        wid = s * 2 + c
        cstart = wid * CPW + jnp.minimum(wid, XTRA)
        extra = wid < XTRA

        _zero_rows(ones_v, CH, 16)
        _zero_acc_slice(ones_v, acc_sh, s)
        pltpu.sync_copy(ei2_hbm.at[1].at[pl.ds(cstart, CPW)],
                        dstr_v.at[pl.ds(0, CPW)])

        @pl.when(extra)
        def _():
            pltpu.sync_copy(ei2_hbm.at[1].at[pl.ds(cstart + CPW, 1)],
                            dstr_v.at[pl.ds(CPW, 1)])

        one = jnp.ones((16,), jnp.float32)

        def fill(i, carry):
            ones_v[i, pl.ds(0, 16)] = one
            return carry

        lax.fori_loop(0, CH, fill, 0)
        plsc.subcore_barrier()

        DGRP = 13

        def body(k, carry):
            for t in range(DGRP):
                pltpu.make_async_copy(
                    ones_v, acc_sh.at[dstr_v.at[k * DGRP + t]],
                    ssem).start(add=True)
            for t in range(DGRP):
                pltpu.make_async_copy(
                    ones_v, acc_sh.at[dstr_v.at[k * DGRP + t]],
                    ssem).wait()
            return carry

        lax.fori_loop(0, CPW // DGRP, body, 0)

        @pl.when(extra)
        def _():
            pltpu.sync_copy(ones_v, acc_sh.at[dstr_v.at[CPW]], add=True)

        plsc.subcore_barrier()
        _copy_out(acc_sh, out_hbm, c, s)

    return deg


_R = 1000


def _tc1_body(degp, x, w1, g1, dinv16):
    deg = degp[0] + degp[1] + 1.0
    dinv = lax.rsqrt(deg)
    dinv16[...] = dinv
    g1[...] = dinv[:, 0:1] * jnp.dot(x[...], w1[...],
                                     preferred_element_type=jnp.float32)


def _tc2_body(aggp, g1, dinv16, b1, w2, g2):
    dinv = dinv16[:, 0:1]
    h = dinv * (aggp[0] + aggp[1] + g1[...]) + b1[...]
    h = jnp.maximum(h, 0.0)
    g2[...] = dinv * jnp.dot(h, w2[...], preferred_element_type=jnp.float32)


def _tc3_body(aggp, g2, dinv16, b2, out):
    dinv = dinv16[:, 0:1]
    logits = dinv * (aggp[0] + aggp[1] + g2[...]) + b2[...]
    m = jnp.max(logits, axis=1, keepdims=True)
    e = logits - m
    out[...] = e - jnp.log(jnp.sum(jnp.exp(e), axis=1, keepdims=True))


def _row_spec(F):
    return pl.BlockSpec((_R, F), lambda i: (i, 0))


def _pair_spec(F):
    return pl.BlockSpec((2, _R, F), lambda i: (0, i, 0))


def _full_spec(a, b):
    return pl.BlockSpec((a, b), lambda i: (0, 0))


_tc1 = pl.pallas_call(
    _tc1_body,
    grid=(N // _R,),
    in_specs=[_pair_spec(16), _row_spec(128), _full_spec(128, 128)],
    out_specs=[_row_spec(128), _row_spec(16)],
    out_shape=[jax.ShapeDtypeStruct((N, 128), jnp.float32),
               jax.ShapeDtypeStruct((N, 16), jnp.float32)],
)

_tc2 = pl.pallas_call(
    _tc2_body,
    grid=(N // _R,),
    in_specs=[_pair_spec(128), _row_spec(128), _row_spec(16),
              _full_spec(1, 128), _full_spec(128, 64)],
    out_specs=[_row_spec(64)],
    out_shape=[jax.ShapeDtypeStruct((N, 64), jnp.float32)],
)

_tc3 = pl.pallas_call(
    _tc3_body,
    grid=(N // _R,),
    in_specs=[_pair_spec(64), _row_spec(64), _row_spec(16),
              _full_spec(1, 64)],
    out_specs=[_row_spec(64)],
    out_shape=[jax.ShapeDtypeStruct((N, 64), jnp.float32)],
)

_agg128 = _make_agg(128)
_agg64 = _make_agg(64)
_deg = _make_deg()


@jax.jit
def kernel(x, edge_index, W1, b1, W2, b2):
    ei2 = edge_index.reshape(2, NCHUNK, CH)
    degp = _deg(ei2)
    g1, dinv16 = _tc1(degp, x, W1)
    aggp1 = _agg128(g1, ei2)
    (g2,) = _tc2(aggp1, g1, dinv16, b1.reshape(1, -1), W2)
    aggp2 = _agg64(g2, ei2)
    (out,) = _tc3(aggp2, g2, dinv16, b2.reshape(1, -1))
    return out


# speedup vs baseline: 35.5882x; 1.0324x over previous
import functools

import jax
import jax.numpy as jnp
from jax import lax
from jax.experimental import pallas as pl
from jax.experimental.pallas import tpu as pltpu
from jax.experimental.pallas import tpu_sc as plsc

N = 10000
E = 320000
CH = 128
NCHUNK = E // CH
NW = 32
CPW = NCHUNK // NW
XTRA = NCHUNK - CPW * NW
GS = 6
NG = CPW // GS
RPS = 624
REM = N - 16 * RPS

_SC_PARAMS = pltpu.CompilerParams(use_tc_tiling_on_sc=False)


def _zero_rows(ref, nrows, ncols):
    z = jnp.zeros((16,), jnp.float32)

    def body(i, carry):
        for k in range(ncols // 16):
            ref[i, pl.ds(k * 16, 16)] = z
        return carry

    lax.fori_loop(0, nrows, body, 0)


def _zero_acc_slice(zbuf, acc_sh, s):
    row0 = s * RPS
    for k in range(4):
        pltpu.sync_copy(zbuf, acc_sh.at[pl.ds(row0 + k * CH, CH)])
    pltpu.sync_copy(zbuf.at[pl.ds(0, RPS - 4 * CH)],
                    acc_sh.at[pl.ds(row0 + 4 * CH, RPS - 4 * CH)])

    @pl.when(s == 15)
    def _():
        pltpu.sync_copy(zbuf.at[pl.ds(0, REM)],
                        acc_sh.at[pl.ds(16 * RPS, REM)])


def _copy_out(acc_sh, out_hbm, c, s):
    row0 = s * RPS
    pltpu.sync_copy(acc_sh.at[pl.ds(row0, RPS)],
                    out_hbm.at[c].at[pl.ds(row0, RPS)])

    @pl.when(s == 15)
    def _():
        pltpu.sync_copy(acc_sh.at[pl.ds(16 * RPS, REM)],
                        out_hbm.at[c].at[pl.ds(16 * RPS, REM)])


def _make_agg(F):
    mesh = plsc.VectorSubcoreMesh(core_axis_name="c", subcore_axis_name="s")

    @functools.partial(
        pl.kernel,
        out_type=jax.ShapeDtypeStruct((2, N, F), jnp.float32),
        mesh=mesh,
        compiler_params=_SC_PARAMS,
        scratch_types=[
            pltpu.VMEM((GS, CH), jnp.int32),
            pltpu.VMEM((GS, CH), jnp.int32),
            pltpu.VMEM((GS, CH), jnp.int32),
            pltpu.VMEM((GS, CH), jnp.int32),
            pltpu.VMEM((CH, F), jnp.float32),
            pltpu.VMEM((CH, F), jnp.float32),
            pltpu.VMEM_SHARED((N, F), jnp.float32),
            pltpu.SemaphoreType.DMA,
            pltpu.SemaphoreType.DMA,
            pltpu.SemaphoreType.DMA,
            pltpu.SemaphoreType.DMA,
        ],
    )
    def agg(g_hbm, ei2_hbm, out_hbm,
            sbuf0, sbuf1, dbuf0, dbuf1, rows0_v, rows1_v, acc_sh,
            gsem0, gsem1, ssem0, ssem1):
        c = lax.axis_index("c")
        s = lax.axis_index("s")
        wid = s * 2 + c
        cstart = wid * CPW + jnp.minimum(wid, XTRA)
        extra = wid < XTRA

        _zero_rows(rows0_v, CH, F)
        _zero_rows(rows1_v, CH, F)
        _zero_acc_slice(rows0_v, acc_sh, s)
        plsc.subcore_barrier()

        sbufs = (sbuf0, sbuf1)
        dbufs = (dbuf0, dbuf1)
        rows = (rows0_v, rows1_v)
        gsems = (gsem0, gsem1)
        ssems = (ssem0, ssem1)

        def load_group(g, p):
            pltpu.sync_copy(ei2_hbm.at[0].at[pl.ds(cstart + g * GS, GS)],
                            sbufs[p])
            pltpu.sync_copy(ei2_hbm.at[1].at[pl.ds(cstart + g * GS, GS)],
                            dbufs[p])

        def gather(p, t, b):
            return pltpu.async_copy(
                g_hbm.at[sbufs[p].at[t]], rows[b], gsems[b])

        def wait_gather(b):
            pltpu.make_async_copy(g_hbm.at[sbufs[0].at[0]],
                                  rows[b], gsems[b]).wait()

        def scatter_start(p, t, b):
            pltpu.make_async_copy(rows[b], acc_sh.at[dbufs[p].at[t]],
                                  ssems[b]).start(add=True)

        def wait_scatter(b):
            pltpu.make_async_copy(rows[b], acc_sh.at[dbufs[0].at[0]],
                                  ssems[b]).wait()

        def process(p, t, nxt):
            b = t % 2
            wait_scatter(1 - b)
            if nxt is not None:
                gather(nxt[0], nxt[1], 1 - b)
            wait_gather(b)
            scatter_start(p, t, b)

        def half_group(p, t0, nxt_last):
            for t in range(t0, t0 + GS // 2):
                nxt = (p, t + 1) if t < GS - 1 else (nxt_last, 0)
                process(p, t, nxt)

        load_group(0, 0)
        pltpu.make_async_copy(rows1_v, acc_sh.at[dbuf0.at[0]],
                              ssem1).start(add=True)
        gather(0, 0, 0)

        def body(k, carry):
            half_group(0, 0, None)
            load_group(2 * k + 1, 1)
            half_group(0, GS // 2, 1)
            half_group(1, 0, None)
            load_group(2 * k + 2, 0)
            half_group(1, GS // 2, 0)
            return carry

        lax.fori_loop(0, NG // 2, body, 0)

        for t in range(GS):
            process(0, t, (0, t + 1) if t < GS - 1 else None)
        wait_scatter(1)

        @pl.when(extra)
        def _():
            pltpu.sync_copy(ei2_hbm.at[0].at[pl.ds(cstart + CPW, 1)],
                            sbufs[1].at[pl.ds(0, 1)])
            pltpu.sync_copy(ei2_hbm.at[1].at[pl.ds(cstart + CPW, 1)],
                            dbufs[1].at[pl.ds(0, 1)])
            gather(1, 0, 0).wait()
            pltpu.sync_copy(rows0_v, acc_sh.at[dbufs[1].at[0]], add=True)

        plsc.subcore_barrier()
        _copy_out(acc_sh, out_hbm, c, s)

    return agg


def _make_deg():
    mesh = plsc.VectorSubcoreMesh(core_axis_name="c", subcore_axis_name="s")

    @functools.partial(
        pl.kernel,
        out_type=jax.ShapeDtypeStruct((2, N, 16), jnp.float32),
        mesh=mesh,
        compiler_params=_SC_PARAMS,
        scratch_types=[
            pltpu.VMEM((CPW + 1, CH), jnp.int32),
            pltpu.VMEM((CH, 16), jnp.float32),
            pltpu.VMEM_SHARED((N, 16), jnp.float32),
            pltpu.SemaphoreType.DMA,
        ],
    )
    def deg(ei2_hbm, out_hbm, dstr_v, ones_v, acc_sh, ssem):
        c = lax.axis_index("c")
        s = lax.axis_index("s")
        wid = s * 2 + c
        cstart = wid * CPW + jnp.minimum(wid, XTRA)
        extra = wid < XTRA

        _zero_rows(ones_v, CH, 16)
        _zero_acc_slice(ones_v, acc_sh, s)
        pltpu.sync_copy(ei2_hbm.at[1].at[pl.ds(cstart, CPW)],
                        dstr_v.at[pl.ds(0, CPW)])

        @pl.when(extra)
        def _():
            pltpu.sync_copy(ei2_hbm.at[1].at[pl.ds(cstart + CPW, 1)],
                            dstr_v.at[pl.ds(CPW, 1)])

        one = jnp.ones((16,), jnp.float32)

        def fill(i, carry):
            ones_v[i, pl.ds(0, 16)] = one
            return carry

        lax.fori_loop(0, CH, fill, 0)
        plsc.subcore_barrier()

        DGRP = 13

        def body(k, carry):
            for t in range(DGRP):
                pltpu.make_async_copy(
                    ones_v, acc_sh.at[dstr_v.at[k * DGRP + t]],
                    ssem).start(add=True)
            for t in range(DGRP):
                pltpu.make_async_copy(
                    ones_v, acc_sh.at[dstr_v.at[k * DGRP + t]],
                    ssem).wait()
            return carry

        lax.fori_loop(0, CPW // DGRP, body, 0)

        @pl.when(extra)
        def _():
            pltpu.sync_copy(ones_v, acc_sh.at[dstr_v.at[CPW]], add=True)

        plsc.subcore_barrier()
        _copy_out(acc_sh, out_hbm, c, s)

    return deg


_R = 1000


def _tc1_body(degp, x, w1, g1, dinv16):
    deg = degp[0] + degp[1] + 1.0
    dinv = lax.rsqrt(deg)
    dinv16[...] = dinv
    g1[...] = dinv[:, 0:1] * jnp.dot(x[...], w1[...],
                                     preferred_element_type=jnp.float32)


def _tc2_body(aggp, g1, dinv16, b1, w2, g2):
    dinv = dinv16[:, 0:1]
    h = dinv * (aggp[0] + aggp[1] + g1[...]) + b1[...]
    h = jnp.maximum(h, 0.0)
    g2[...] = dinv * jnp.dot(h, w2[...], preferred_element_type=jnp.float32)


def _tc3_body(aggp, g2, dinv16, b2, out):
    dinv = dinv16[:, 0:1]
    logits = dinv * (aggp[0] + aggp[1] + g2[...]) + b2[...]
    m = jnp.max(logits, axis=1, keepdims=True)
    e = logits - m
    out[...] = e - jnp.log(jnp.sum(jnp.exp(e), axis=1, keepdims=True))


def _row_spec(F):
    return pl.BlockSpec((_R, F), lambda i: (i, 0))


def _pair_spec(F):
    return pl.BlockSpec((2, _R, F), lambda i: (0, i, 0))


def _full_spec(a, b):
    return pl.BlockSpec((a, b), lambda i: (0, 0))


_tc1 = pl.pallas_call(
    _tc1_body,
    grid=(N // _R,),
    in_specs=[_pair_spec(16), _row_spec(128), _full_spec(128, 128)],
    out_specs=[_row_spec(128), _row_spec(16)],
    out_shape=[jax.ShapeDtypeStruct((N, 128), jnp.float32),
               jax.ShapeDtypeStruct((N, 16), jnp.float32)],
)

_tc2 = pl.pallas_call(
    _tc2_body,
    grid=(N // _R,),
    in_specs=[_pair_spec(128), _row_spec(128), _row_spec(16),
              _full_spec(1, 128), _full_spec(128, 64)],
    out_specs=[_row_spec(64)],
    out_shape=[jax.ShapeDtypeStruct((N, 64), jnp.float32)],
)

_tc3 = pl.pallas_call(
    _tc3_body,
    grid=(N // _R,),
    in_specs=[_pair_spec(64), _row_spec(64), _row_spec(16),
              _full_spec(1, 64)],
    out_specs=[_row_spec(64)],
    out_shape=[jax.ShapeDtypeStruct((N, 64), jnp.float32)],
)

_agg128 = _make_agg(128)
_agg64 = _make_agg(64)
_deg = _make_deg()


@jax.jit
def kernel(x, edge_index, W1, b1, W2, b2):
    ei2 = edge_index.reshape(2, NCHUNK, CH)
    degp = _deg(ei2)
    g1, dinv16 = _tc1(degp, x, W1)
    aggp1 = _agg128(g1, ei2)
    (g2,) = _tc2(aggp1, g1, dinv16, b1.reshape(1, -1), W2)
    aggp2 = _agg64(g2, ei2)
    (out,) = _tc3(aggp2, g2, dinv16, b2.reshape(1, -1))
    return out


# speedup vs baseline: 36.0234x; 1.0122x over previous
import functools

import jax
import jax.numpy as jnp
from jax import lax
from jax.experimental import pallas as pl
from jax.experimental.pallas import tpu as pltpu
from jax.experimental.pallas import tpu_sc as plsc

N = 10000
E = 320000
CH = 128
NCHUNK = E // CH
NW = 32
CPW = NCHUNK // NW
XTRA = NCHUNK - CPW * NW
GS = 6
NG = CPW // GS
RPS = 624
REM = N - 16 * RPS

_SC_PARAMS = pltpu.CompilerParams(use_tc_tiling_on_sc=False)


def _zero_rows(ref, nrows, ncols):
    z = jnp.zeros((16,), jnp.float32)

    def body(i, carry):
        for k in range(ncols // 16):
            ref[i, pl.ds(k * 16, 16)] = z
        return carry

    lax.fori_loop(0, nrows, body, 0)


def _zero_acc_slice(zbuf, acc_sh, s):
    row0 = s * RPS
    for k in range(4):
        pltpu.sync_copy(zbuf, acc_sh.at[pl.ds(row0 + k * CH, CH)])
    pltpu.sync_copy(zbuf.at[pl.ds(0, RPS - 4 * CH)],
                    acc_sh.at[pl.ds(row0 + 4 * CH, RPS - 4 * CH)])

    @pl.when(s == 15)
    def _():
        pltpu.sync_copy(zbuf.at[pl.ds(0, REM)],
                        acc_sh.at[pl.ds(16 * RPS, REM)])


def _copy_out(acc_sh, out_hbm, c, s):
    row0 = s * RPS
    pltpu.sync_copy(acc_sh.at[pl.ds(row0, RPS)],
                    out_hbm.at[c].at[pl.ds(row0, RPS)])

    @pl.when(s == 15)
    def _():
        pltpu.sync_copy(acc_sh.at[pl.ds(16 * RPS, REM)],
                        out_hbm.at[c].at[pl.ds(16 * RPS, REM)])


def _make_agg(F, NBUF):
    mesh = plsc.VectorSubcoreMesh(core_axis_name="c", subcore_axis_name="s")

    @functools.partial(
        pl.kernel,
        out_type=jax.ShapeDtypeStruct((2, N, F), jnp.float32),
        mesh=mesh,
        compiler_params=_SC_PARAMS,
        scratch_types=(
            [pltpu.VMEM((GS, CH), jnp.int32)] * 4
            + [pltpu.VMEM((CH, F), jnp.float32)] * NBUF
            + [pltpu.VMEM_SHARED((N, F), jnp.float32)]
            + [pltpu.SemaphoreType.DMA] * (2 * NBUF)
        ),
    )
    def agg(g_hbm, ei2_hbm, out_hbm, *scr):
        sbufs = scr[0:2]
        dbufs = scr[2:4]
        rows = scr[4:4 + NBUF]
        acc_sh = scr[4 + NBUF]
        gsems = scr[5 + NBUF:5 + 2 * NBUF]
        ssems = scr[5 + 2 * NBUF:5 + 3 * NBUF]
        c = lax.axis_index("c")
        s = lax.axis_index("s")
        wid = s * 2 + c
        cstart = wid * CPW + jnp.minimum(wid, XTRA)
        extra = wid < XTRA

        for r in rows:
            _zero_rows(r, CH, F)
        _zero_acc_slice(rows[0], acc_sh, s)
        plsc.subcore_barrier()

        def load_group(g, p):
            pltpu.sync_copy(ei2_hbm.at[0].at[pl.ds(cstart + g * GS, GS)],
                            sbufs[p])
            pltpu.sync_copy(ei2_hbm.at[1].at[pl.ds(cstart + g * GS, GS)],
                            dbufs[p])

        def gather(p, t, b):
            return pltpu.async_copy(
                g_hbm.at[sbufs[p].at[t]], rows[b], gsems[b])

        def wait_gather(b):
            pltpu.make_async_copy(g_hbm.at[sbufs[0].at[0]],
                                  rows[b], gsems[b]).wait()

        def scatter_start(p, t, b):
            pltpu.make_async_copy(rows[b], acc_sh.at[dbufs[p].at[t]],
                                  ssems[b]).start(add=True)

        def wait_scatter(b):
            pltpu.make_async_copy(rows[b], acc_sh.at[dbufs[0].at[0]],
                                  ssems[b]).wait()

        def process(p, t, o, nxt):
            b = o % NBUF
            nb = (o + 1) % NBUF
            wait_scatter(nb)
            if nxt is not None:
                gather(nxt[0], nxt[1], nb)
            wait_gather(b)
            scatter_start(p, t, b)

        def half_group(p, t0, o0, nxt_last):
            for t in range(t0, t0 + GS // 2):
                nxt = (p, t + 1) if t < GS - 1 else (nxt_last, 0)
                process(p, t, o0 + t - t0, nxt)

        load_group(0, 0)
        for b in range(1, NBUF):
            pltpu.make_async_copy(rows[b], acc_sh.at[dbufs[0].at[0]],
                                  ssems[b]).start(add=True)
        gather(0, 0, 0)

        def body(k, carry):
            half_group(0, 0, 0, None)
            load_group(2 * k + 1, 1)
            half_group(0, GS // 2, GS // 2, 1)
            half_group(1, 0, GS, None)
            load_group(2 * k + 2, 0)
            half_group(1, GS // 2, GS + GS // 2, 0)
            return carry

        lax.fori_loop(0, NG // 2, body, 0)

        for t in range(GS):
            process(0, t, t, (0, t + 1) if t < GS - 1 else None)
        starts = [CPW // NBUF + (1 if k < CPW % NBUF else 0) + (k >= 1)
                  for k in range(NBUF)]
        waits = [sum(1 for j in range(CPW) if (j + 1) % NBUF == k)
                 for k in range(NBUF)]
        for k in range(NBUF):
            for _ in range(starts[k] - waits[k]):
                wait_scatter(k)

        @pl.when(extra)
        def _():
            pltpu.sync_copy(ei2_hbm.at[0].at[pl.ds(cstart + CPW, 1)],
                            sbufs[1].at[pl.ds(0, 1)])
            pltpu.sync_copy(ei2_hbm.at[1].at[pl.ds(cstart + CPW, 1)],
                            dbufs[1].at[pl.ds(0, 1)])
            gather(1, 0, 0).wait()
            pltpu.sync_copy(rows[0], acc_sh.at[dbufs[1].at[0]], add=True)

        plsc.subcore_barrier()
        _copy_out(acc_sh, out_hbm, c, s)

    return agg


def _make_deg():
    mesh = plsc.VectorSubcoreMesh(core_axis_name="c", subcore_axis_name="s")

    @functools.partial(
        pl.kernel,
        out_type=jax.ShapeDtypeStruct((2, N, 16), jnp.float32),
        mesh=mesh,
        compiler_params=_SC_PARAMS,
        scratch_types=[
            pltpu.VMEM((CPW + 1, CH), jnp.int32),
            pltpu.VMEM((CH, 16), jnp.float32),
            pltpu.VMEM_SHARED((N, 16), jnp.float32),
            pltpu.SemaphoreType.DMA,
        ],
    )
    def deg(ei2_hbm, out_hbm, dstr_v, ones_v, acc_sh, ssem):
        c = lax.axis_index("c")
        s = lax.axis_index("s")
        wid = s * 2 + c
        cstart = wid * CPW + jnp.minimum(wid, XTRA)
        extra = wid < XTRA

        _zero_rows(ones_v, CH, 16)
        _zero_acc_slice(ones_v, acc_sh, s)
        pltpu.sync_copy(ei2_hbm.at[1].at[pl.ds(cstart, CPW)],
                        dstr_v.at[pl.ds(0, CPW)])

        @pl.when(extra)
        def _():
            pltpu.sync_copy(ei2_hbm.at[1].at[pl.ds(cstart + CPW, 1)],
                            dstr_v.at[pl.ds(CPW, 1)])

        one = jnp.ones((16,), jnp.float32)

        def fill(i, carry):
            ones_v[i, pl.ds(0, 16)] = one
            return carry

        lax.fori_loop(0, CH, fill, 0)
        plsc.subcore_barrier()

        DGRP = 13

        def body(k, carry):
            for t in range(DGRP):
                pltpu.make_async_copy(
                    ones_v, acc_sh.at[dstr_v.at[k * DGRP + t]],
                    ssem).start(add=True)
            for t in range(DGRP):
                pltpu.make_async_copy(
                    ones_v, acc_sh.at[dstr_v.at[k * DGRP + t]],
                    ssem).wait()
            return carry

        lax.fori_loop(0, CPW // DGRP, body, 0)

        @pl.when(extra)
        def _():
            pltpu.sync_copy(ones_v, acc_sh.at[dstr_v.at[CPW]], add=True)

        plsc.subcore_barrier()
        _copy_out(acc_sh, out_hbm, c, s)

    return deg


_R = 2000


def _tc1_body(degp, x, w1, g1, dinv16):
    deg = degp[0] + degp[1] + 1.0
    dinv = lax.rsqrt(deg)
    dinv16[...] = dinv
    g1[...] = dinv[:, 0:1] * jnp.dot(x[...], w1[...],
                                     preferred_element_type=jnp.float32)


def _tc2_body(aggp, g1, dinv16, b1, w2, g2):
    dinv = dinv16[:, 0:1]
    h = dinv * (aggp[0] + aggp[1] + g1[...]) + b1[...]
    h = jnp.maximum(h, 0.0)
    g2[...] = dinv * jnp.dot(h, w2[...], preferred_element_type=jnp.float32)


def _tc3_body(aggp, g2, dinv16, b2, out):
    dinv = dinv16[:, 0:1]
    logits = dinv * (aggp[0] + aggp[1] + g2[...]) + b2[...]
    m = jnp.max(logits, axis=1, keepdims=True)
    e = logits - m
    out[...] = e - jnp.log(jnp.sum(jnp.exp(e), axis=1, keepdims=True))


def _row_spec(F):
    return pl.BlockSpec((_R, F), lambda i: (i, 0))


def _pair_spec(F):
    return pl.BlockSpec((2, _R, F), lambda i: (0, i, 0))


def _full_spec(a, b):
    return pl.BlockSpec((a, b), lambda i: (0, 0))


_tc1 = pl.pallas_call(
    _tc1_body,
    grid=(N // _R,),
    in_specs=[_pair_spec(16), _row_spec(128), _full_spec(128, 128)],
    out_specs=[_row_spec(128), _row_spec(16)],
    out_shape=[jax.ShapeDtypeStruct((N, 128), jnp.float32),
               jax.ShapeDtypeStruct((N, 16), jnp.float32)],
)

_tc2 = pl.pallas_call(
    _tc2_body,
    grid=(N // _R,),
    in_specs=[_pair_spec(128), _row_spec(128), _row_spec(16),
              _full_spec(1, 128), _full_spec(128, 64)],
    out_specs=[_row_spec(64)],
    out_shape=[jax.ShapeDtypeStruct((N, 64), jnp.float32)],
)

_tc3 = pl.pallas_call(
    _tc3_body,
    grid=(N // _R,),
    in_specs=[_pair_spec(64), _row_spec(64), _row_spec(16),
              _full_spec(1, 64)],
    out_specs=[_row_spec(64)],
    out_shape=[jax.ShapeDtypeStruct((N, 64), jnp.float32)],
)

_agg128 = _make_agg(128, 2)
_agg64 = _make_agg(64, 4)
_deg = _make_deg()


@jax.jit
def kernel(x, edge_index, W1, b1, W2, b2):
    ei2 = edge_index.reshape(2, NCHUNK, CH)
    degp = _deg(ei2)
    g1, dinv16 = _tc1(degp, x, W1)
    aggp1 = _agg128(g1, ei2)
    (g2,) = _tc2(aggp1, g1, dinv16, b1.reshape(1, -1), W2)
    aggp2 = _agg64(g2, ei2)
    (out,) = _tc3(aggp2, g2, dinv16, b2.reshape(1, -1))
    return out


# speedup vs baseline: 37.0948x; 1.0297x over previous
import functools

import jax
import jax.numpy as jnp
from jax import lax
from jax.experimental import pallas as pl
from jax.experimental.pallas import tpu as pltpu
from jax.experimental.pallas import tpu_sc as plsc

N = 10000
E = 320000
CH = 128
NCHUNK = E // CH
NW = 32
CPW = NCHUNK // NW
XTRA = NCHUNK - CPW * NW
GS = 6
NG = CPW // GS
RPS = 624
REM = N - 16 * RPS

_SC_PARAMS = pltpu.CompilerParams(use_tc_tiling_on_sc=False)


def _zero_rows(ref, nrows, ncols):
    z = jnp.zeros((16,), jnp.float32)

    def body(i, carry):
        for k in range(ncols // 16):
            ref[i, pl.ds(k * 16, 16)] = z
        return carry

    lax.fori_loop(0, nrows, body, 0)


def _zero_acc_slice(zbuf, acc_sh, s):
    row0 = s * RPS
    for k in range(4):
        pltpu.sync_copy(zbuf, acc_sh.at[pl.ds(row0 + k * CH, CH)])
    pltpu.sync_copy(zbuf.at[pl.ds(0, RPS - 4 * CH)],
                    acc_sh.at[pl.ds(row0 + 4 * CH, RPS - 4 * CH)])

    @pl.when(s == 15)
    def _():
        pltpu.sync_copy(zbuf.at[pl.ds(0, REM)],
                        acc_sh.at[pl.ds(16 * RPS, REM)])


def _copy_out(acc_sh, out_hbm, c, s):
    row0 = s * RPS
    pltpu.sync_copy(acc_sh.at[pl.ds(row0, RPS)],
                    out_hbm.at[c].at[pl.ds(row0, RPS)])

    @pl.when(s == 15)
    def _():
        pltpu.sync_copy(acc_sh.at[pl.ds(16 * RPS, REM)],
                        out_hbm.at[c].at[pl.ds(16 * RPS, REM)])


def _make_agg(F, NBUF):
    mesh = plsc.VectorSubcoreMesh(core_axis_name="c", subcore_axis_name="s")

    @functools.partial(
        pl.kernel,
        out_type=jax.ShapeDtypeStruct((2, N, F), jnp.float32),
        mesh=mesh,
        compiler_params=_SC_PARAMS,
        scratch_types=(
            [pltpu.VMEM((GS, CH), jnp.int32)] * 4
            + [pltpu.VMEM((CH, F), jnp.float32)] * NBUF
            + [pltpu.VMEM_SHARED((N, F), jnp.float32)]
            + [pltpu.SemaphoreType.DMA] * (2 * NBUF)
        ),
    )
    def agg(g_hbm, ei2_hbm, out_hbm, *scr):
        sbufs = scr[0:2]
        dbufs = scr[2:4]
        rows = scr[4:4 + NBUF]
        acc_sh = scr[4 + NBUF]
        gsems = scr[5 + NBUF:5 + 2 * NBUF]
        ssems = scr[5 + 2 * NBUF:5 + 3 * NBUF]
        c = lax.axis_index("c")
        s = lax.axis_index("s")
        wid = s * 2 + c
        cstart = wid * CPW + jnp.minimum(wid, XTRA)
        extra = wid < XTRA

        for r in rows:
            _zero_rows(r, CH, F)
        _zero_acc_slice(rows[0], acc_sh, s)
        plsc.subcore_barrier()

        def load_group(g, p):
            pltpu.sync_copy(ei2_hbm.at[0].at[pl.ds(cstart + g * GS, GS)],
                            sbufs[p])
            pltpu.sync_copy(ei2_hbm.at[1].at[pl.ds(cstart + g * GS, GS)],
                            dbufs[p])

        def gather(p, t, b):
            return pltpu.async_copy(
                g_hbm.at[sbufs[p].at[t]], rows[b], gsems[b])

        def wait_gather(b):
            pltpu.make_async_copy(g_hbm.at[sbufs[0].at[0]],
                                  rows[b], gsems[b]).wait()

        def scatter_start(p, t, b):
            pltpu.make_async_copy(rows[b], acc_sh.at[dbufs[p].at[t]],
                                  ssems[b]).start(add=True)

        def wait_scatter(b):
            pltpu.make_async_copy(rows[b], acc_sh.at[dbufs[0].at[0]],
                                  ssems[b]).wait()

        def process(p, t, o, nxt):
            b = o % NBUF
            nb = (o + 1) % NBUF
            wait_scatter(nb)
            if nxt is not None:
                gather(nxt[0], nxt[1], nb)
            wait_gather(b)
            scatter_start(p, t, b)

        def half_group(p, t0, o0, nxt_last):
            for t in range(t0, t0 + GS // 2):
                nxt = (p, t + 1) if t < GS - 1 else (nxt_last, 0)
                process(p, t, o0 + t - t0, nxt)

        load_group(0, 0)
        for b in range(1, NBUF):
            pltpu.make_async_copy(rows[b], acc_sh.at[dbufs[0].at[0]],
                                  ssems[b]).start(add=True)
        gather(0, 0, 0)

        def body(k, carry):
            half_group(0, 0, 0, None)
            load_group(2 * k + 1, 1)
            half_group(0, GS // 2, GS // 2, 1)
            half_group(1, 0, GS, None)
            load_group(2 * k + 2, 0)
            half_group(1, GS // 2, GS + GS // 2, 0)
            return carry

        lax.fori_loop(0, NG // 2, body, 0)

        for t in range(GS):
            process(0, t, t, (0, t + 1) if t < GS - 1 else None)
        starts = [CPW // NBUF + (1 if k < CPW % NBUF else 0) + (k >= 1)
                  for k in range(NBUF)]
        waits = [sum(1 for j in range(CPW) if (j + 1) % NBUF == k)
                 for k in range(NBUF)]
        for k in range(NBUF):
            for _ in range(starts[k] - waits[k]):
                wait_scatter(k)

        @pl.when(extra)
        def _():
            pltpu.sync_copy(ei2_hbm.at[0].at[pl.ds(cstart + CPW, 1)],
                            sbufs[1].at[pl.ds(0, 1)])
            pltpu.sync_copy(ei2_hbm.at[1].at[pl.ds(cstart + CPW, 1)],
                            dbufs[1].at[pl.ds(0, 1)])
            gather(1, 0, 0).wait()
            pltpu.sync_copy(rows[0], acc_sh.at[dbufs[1].at[0]], add=True)

        plsc.subcore_barrier()
        _copy_out(acc_sh, out_hbm, c, s)

    return agg


def _make_deg():
    mesh = plsc.VectorSubcoreMesh(core_axis_name="c", subcore_axis_name="s")

    @functools.partial(
        pl.kernel,
        out_type=jax.ShapeDtypeStruct((2, N, 16), jnp.float32),
        mesh=mesh,
        compiler_params=_SC_PARAMS,
        scratch_types=[
            pltpu.VMEM((CPW + 1, CH), jnp.int32),
            pltpu.VMEM((CH, 16), jnp.float32),
            pltpu.VMEM_SHARED((N, 16), jnp.float32),
            pltpu.SemaphoreType.DMA,
        ],
    )
    def deg(ei2_hbm, out_hbm, dstr_v, ones_v, acc_sh, ssem):
        c = lax.axis_index("c")
        s = lax.axis_index("s")
        wid = s * 2 + c
        cstart = wid * CPW + jnp.minimum(wid, XTRA)
        extra = wid < XTRA

        _zero_rows(ones_v, CH, 16)
        _zero_acc_slice(ones_v, acc_sh, s)
        pltpu.sync_copy(ei2_hbm.at[1].at[pl.ds(cstart, CPW)],
                        dstr_v.at[pl.ds(0, CPW)])

        @pl.when(extra)
        def _():
            pltpu.sync_copy(ei2_hbm.at[1].at[pl.ds(cstart + CPW, 1)],
                            dstr_v.at[pl.ds(CPW, 1)])

        one = jnp.ones((16,), jnp.float32)

        def fill(i, carry):
            ones_v[i, pl.ds(0, 16)] = one
            return carry

        lax.fori_loop(0, CH, fill, 0)
        plsc.subcore_barrier()

        DGRP = 13

        def body(k, carry):
            for t in range(DGRP):
                pltpu.make_async_copy(
                    ones_v, acc_sh.at[dstr_v.at[k * DGRP + t]],
                    ssem).start(add=True)
            for t in range(DGRP):
                pltpu.make_async_copy(
                    ones_v, acc_sh.at[dstr_v.at[k * DGRP + t]],
                    ssem).wait()
            return carry

        lax.fori_loop(0, CPW // DGRP, body, 0)

        @pl.when(extra)
        def _():
            pltpu.sync_copy(ones_v, acc_sh.at[dstr_v.at[CPW]], add=True)

        plsc.subcore_barrier()
        _copy_out(acc_sh, out_hbm, c, s)

    return deg


_R = 2000
_RP = _R // 8


def _dsel(dinvp):
    l = lax.broadcasted_iota(jnp.int32, (128, 8), 0)
    k = lax.broadcasted_iota(jnp.int32, (128, 8), 1)
    sel = jnp.where(l == 16 * k, 1.0, 0.0).astype(jnp.float32)
    return jnp.dot(dinvp, sel, preferred_element_type=jnp.float32)


def _rowscale(dsel, t):
    t3 = t.reshape(_RP, 8, t.shape[-1])
    return (dsel[:, :, None] * t3).reshape(t.shape)


def _tc1_body(degp, x, w1, g1, dinvp_out):
    r0 = pl.program_id(0) * _RP
    dinvp = lax.rsqrt(degp[0, pl.ds(r0, _RP), :]
                      + degp[1, pl.ds(r0, _RP), :] + 1.0)
    dinvp_out[pl.ds(r0, _RP), :] = dinvp
    u1 = jnp.dot(x[...], w1[...], preferred_element_type=jnp.float32)
    g1[...] = _rowscale(_dsel(dinvp), u1)


def _tc2_body(aggp, g1, dinvp, b1, w2, g2):
    r0 = pl.program_id(0) * _RP
    dsel = _dsel(dinvp[pl.ds(r0, _RP), :])
    h = _rowscale(dsel, aggp[0] + aggp[1] + g1[...]) + b1[...]
    h = jnp.maximum(h, 0.0)
    g2[...] = _rowscale(dsel, jnp.dot(h, w2[...],
                                      preferred_element_type=jnp.float32))


def _tc3_body(aggp, g2, dinvp, b2, out):
    r0 = pl.program_id(0) * _RP
    logits = _rowscale(_dsel(dinvp[pl.ds(r0, _RP), :]),
                       aggp[0] + aggp[1] + g2[...]) + b2[...]
    m = jnp.max(logits, axis=1, keepdims=True)
    e = logits - m
    out[...] = e - jnp.log(jnp.sum(jnp.exp(e), axis=1, keepdims=True))


def _row_spec(F):
    return pl.BlockSpec((_R, F), lambda i: (i, 0))


def _pair_spec(F):
    return pl.BlockSpec((2, _R, F), lambda i: (0, i, 0))


def _full_spec(a, b):
    return pl.BlockSpec((a, b), lambda i: (0, 0))


_packed_spec = pl.BlockSpec((N // 8, 128), lambda i: (0, 0))
_pairp_spec = pl.BlockSpec((2, N // 8, 128), lambda i: (0, 0, 0))

_tc1 = pl.pallas_call(
    _tc1_body,
    grid=(N // _R,),
    in_specs=[_pairp_spec, _row_spec(128), _full_spec(128, 128)],
    out_specs=[_row_spec(128), _packed_spec],
    out_shape=[jax.ShapeDtypeStruct((N, 128), jnp.float32),
               jax.ShapeDtypeStruct((N // 8, 128), jnp.float32)],
)

_tc2 = pl.pallas_call(
    _tc2_body,
    grid=(N // _R,),
    in_specs=[_pair_spec(128), _row_spec(128), _packed_spec,
              _full_spec(1, 128), _full_spec(128, 64)],
    out_specs=[_row_spec(64)],
    out_shape=[jax.ShapeDtypeStruct((N, 64), jnp.float32)],
)

_tc3 = pl.pallas_call(
    _tc3_body,
    grid=(N // _R,),
    in_specs=[_pair_spec(64), _row_spec(64), _packed_spec,
              _full_spec(1, 64)],
    out_specs=[_row_spec(64)],
    out_shape=[jax.ShapeDtypeStruct((N, 64), jnp.float32)],
)

_agg128 = _make_agg(128, 2)
_agg64 = _make_agg(64, 4)
_deg = _make_deg()


@jax.jit
def kernel(x, edge_index, W1, b1, W2, b2):
    ei2 = edge_index.reshape(2, NCHUNK, CH)
    degp = _deg(ei2).reshape(2, N // 8, 128)
    g1, dinvp = _tc1(degp, x, W1)
    aggp1 = _agg128(g1, ei2)
    (g2,) = _tc2(aggp1, g1, dinvp, b1.reshape(1, -1), W2)
    aggp2 = _agg64(g2, ei2)
    (out,) = _tc3(aggp2, g2, dinvp, b2.reshape(1, -1))
    return out
